# Initial kernel scaffold; baseline (speedup 1.0000x reference)
import jax
import jax.numpy as jnp
from jax.experimental import pallas as pl


def kernel(embeds, flat_info, cards_rank, cards_suit, cards_enhancement, W0, b0, W1, b1, W2, b2, W3, b3, A0, a0, A1, a1, A2, a2):
    raise NotImplementedError("write your pallas kernel here")


# speedup vs baseline: 7.4956x; 7.4956x over previous
from itertools import combinations

import numpy as np
import jax
import jax.numpy as jnp
from jax.experimental import pallas as pl

_NCARDS = 8
_M = 218
_MP = 224
_TB = 8
_ROWS = _TB * _MP


def _build_masks():
    rows = []
    for n in range(1, 6):
        for combo in combinations(range(_NCARDS), n):
            m = np.zeros(_NCARDS, dtype=np.float32)
            m[list(combo)] = 1.0
            rows.append(m)
    return np.stack(rows, axis=0)


_MASKS = np.zeros((_MP, _NCARDS), dtype=np.float32)
_MASKS[:_M] = _build_masks()
_MASKS_TILED = np.tile(_MASKS, (_TB, 1))
_SIZES_TILED = np.tile(_MASKS.sum(1), _TB)[:, None]

_STRAIGHT = np.zeros((16, 14), dtype=np.float32)
_STRAIGHT[:10] = np.array([
    [0,1,1,1,1,1,0,0,0,0,0,0,0,0],
    [0,0,1,1,1,1,1,0,0,0,0,0,0,0],
    [0,0,0,1,1,1,1,1,0,0,0,0,0,0],
    [0,0,0,0,1,1,1,1,1,0,0,0,0,0],
    [0,0,0,0,0,1,1,1,1,1,0,0,0,0],
    [0,0,0,0,0,0,1,1,1,1,1,0,0,0],
    [0,0,0,0,0,0,0,1,1,1,1,1,0,0],
    [0,0,0,0,0,0,0,0,1,1,1,1,1,0],
    [0,0,0,0,0,0,0,0,0,1,1,1,1,1],
    [0,1,1,1,1,0,0,0,0,0,0,0,0,1]], dtype=np.float32)


def _gelu(x):
    return x * 0.5 * (1.0 + jax.lax.erf(x * np.float32(1.0 / np.sqrt(2.0))))


def _body(rank_ref, suit_ref, enh_ref, flat_ref, masks_ref, sizes_ref,
          str_ref, w0f_ref, w0md_ref, b0_ref, w1_ref, b1_ref, w2_ref, b2_ref,
          w3_ref, b3_ref, a0f_ref, a0md_ref, a0b_ref, a1_ref, a1b_ref,
          a2_ref, a2b_ref, log_ref, aux_ref):
    f32 = jnp.float32

    def to_rows(x):
        return jnp.broadcast_to(x[:, None, :], (_TB, _MP, x.shape[-1])).reshape(_ROWS, x.shape[-1])

    rank = to_rows(rank_ref[...])
    suit = to_rows(suit_ref[...])
    enh = to_rows(enh_ref[...])
    masks = masks_ref[...]

    iota14 = jax.lax.broadcasted_iota(jnp.int32, (1, 14), 1)
    iota6 = jax.lax.broadcasted_iota(jnp.int32, (1, 6), 1)

    rc_in = jnp.zeros((_ROWS, 14), f32)
    rc_tot = jnp.zeros((_ROWS, 14), f32)
    sc_in = jnp.zeros((_ROWS, 6), f32)
    sc_tot = jnp.zeros((_ROWS, 6), f32)
    invc = jnp.zeros((_ROWS, 1), f32)
    for h in range(_NCARDS):
        mh = masks[:, h:h + 1]
        rh = rank[:, h:h + 1]
        sh = suit[:, h:h + 1]
        oh_r = (rh == iota14).astype(f32)
        oh_s = (sh == iota6).astype(f32)
        rc_in = rc_in + mh * oh_r
        rc_tot = rc_tot + oh_r
        sc_in = sc_in + mh * oh_s
        sc_tot = sc_tot + oh_s
        inv_h = ((rh == 0) & (sh == 0) & (enh[:, h:h + 1] == 0)).astype(f32)
        invc = invc + mh * inv_h

    rc_out = rc_tot - rc_in
    sc_out = sc_tot - sc_in
    nz14 = (iota14 != 0).astype(f32)
    nz6 = (iota6 != 0).astype(f32)
    rc_in = rc_in * nz14
    rc_out = rc_out * nz14
    sc_in = sc_in * nz6
    sc_out = sc_out * nz6

    sizes_in = sizes_ref[...]
    sizes_out = np.float32(_NCARDS) - sizes_in

    def metadata(rc, sc, sizes):
        msr = rc.max(axis=-1, keepdims=True)
        distinct = (rc > 0).astype(f32).sum(axis=-1, keepdims=True)
        mss = sc.max(axis=-1, keepdims=True) + sc[:, 5:6]
        hits = jax.lax.dot_general(rc, str_ref[...], (((1,), (1,)), ((), ())),
                                   preferred_element_type=f32)
        msh = hits.max(axis=-1, keepdims=True)
        is_straight = (sizes >= 5) & (msh == 5) & (distinct >= 5)
        is_flush = (sizes >= 5) & (mss >= 5)
        is_sf = is_straight & is_flush
        has_pair = (sizes >= 2) & (msr >= 2)
        has_three = (sizes >= 3) & (msr >= 3)
        has_four = (sizes >= 4) & (msr >= 4)
        has_fh = (sizes >= 5) & has_three & (~has_four) & (distinct == 2)
        num_pairs = (rc >= 2).astype(f32).sum(axis=-1, keepdims=True)
        has_two_pair = (sizes >= 4) & (num_pairs == 2)
        flags = jnp.concatenate(
            [b.astype(f32) for b in
             (has_pair, has_three, has_four, has_fh, has_two_pair,
              is_straight, is_flush, is_sf)], axis=-1)
        extra = jnp.concatenate([msr, mss, distinct, msh, sizes], axis=-1)
        return jnp.concatenate([flags, extra, rc, sc], axis=-1)

    md = jnp.concatenate(
        [metadata(rc_in, sc_in, sizes_in),
         metadata(rc_out, sc_out, sizes_out),
         jnp.zeros((_ROWS, 6), f32)], axis=-1)

    flat = flat_ref[...]
    yh = to_rows(jnp.dot(flat, w0f_ref[...], preferred_element_type=f32))
    yg = to_rows(jnp.dot(flat, a0f_ref[...], preferred_element_type=f32))

    h0 = _gelu(jnp.dot(md, w0md_ref[...], preferred_element_type=f32)
               + yh + b0_ref[...])
    h1 = _gelu(jnp.dot(h0, w1_ref[...], preferred_element_type=f32) + b1_ref[...])
    h2 = _gelu(jnp.dot(h1, w2_ref[...], preferred_element_type=f32) + b2_ref[...])
    logit = (h2 * w3_ref[...]).sum(axis=-1, keepdims=True) + b3_ref[...]

    g0 = _gelu(jnp.dot(md, a0md_ref[...], preferred_element_type=f32)
               + yg + a0b_ref[...])
    g1 = _gelu(jnp.dot(g0, a1_ref[...], preferred_element_type=f32) + a1b_ref[...])
    aux = (g1 * a2_ref[...]).sum(axis=-1, keepdims=True) + a2b_ref[...]

    valid = invc == 0.0
    log_ref[...] = jnp.where(valid, logit, -1e9)
    aux_ref[...] = jnp.where(valid, aux, 0.0)


def kernel(embeds, flat_info, cards_rank, cards_suit, cards_enhancement,
           W0, b0, W1, b1, W2, b2, W3, b3, A0, a0, A1, a1, A2, a2):
    del embeds
    B = flat_info.shape[0]
    f32 = jnp.float32
    rank = cards_rank.astype(jnp.int32)
    suit = cards_suit.astype(jnp.int32)
    enh = cards_enhancement.astype(jnp.int32)

    masks_tiled = jnp.asarray(_MASKS_TILED)
    sizes_tiled = jnp.asarray(_SIZES_TILED)
    straight = jnp.asarray(_STRAIGHT)

    flat_w = int(flat_info.shape[-1])
    W0f, W0md_raw = W0[:flat_w], W0[flat_w:]
    A0f, A0md_raw = A0[:flat_w], A0[flat_w:]
    W0md = jnp.concatenate([W0md_raw, jnp.zeros((6, W0.shape[1]), f32)], axis=0)
    A0md = jnp.concatenate([A0md_raw, jnp.zeros((6, A0.shape[1]), f32)], axis=0)

    grid = B // _TB

    def tiled(shape):
        return pl.BlockSpec(shape, lambda i: (i, 0))

    def whole(x):
        return pl.BlockSpec(x.shape, lambda i: (0,) * x.ndim)

    operands = (
        rank, suit, enh, flat_info, masks_tiled, sizes_tiled, straight,
        W0f, W0md, b0[None], W1, b1[None], W2, b2[None],
        W3.T, b3[None], A0f, A0md, a0[None], A1, a1[None], A2.T, a2[None],
    )
    in_specs = [
        tiled((_TB, _NCARDS)), tiled((_TB, _NCARDS)), tiled((_TB, _NCARDS)),
        tiled((_TB, flat_w)),
    ] + [whole(x) for x in operands[4:]]

    out_shape = [jax.ShapeDtypeStruct((B * _MP, 1), f32)] * 2
    out_specs = [pl.BlockSpec((_ROWS, 1), lambda i: (i, 0))] * 2

    logits_col, aux_col = pl.pallas_call(
        _body,
        grid=(grid,),
        in_specs=in_specs,
        out_specs=out_specs,
        out_shape=out_shape,
    )(*operands)

    logits = logits_col.reshape(B, _MP)[:, :_M]
    aux = aux_col.reshape(B, _MP)[:, :_M, None]
    return logits, aux


# speedup vs baseline: 10.0829x; 1.3452x over previous
from itertools import combinations

import numpy as np
import jax
import jax.numpy as jnp
from jax.experimental import pallas as pl

_NCARDS = 8
_M = 218
_MP = 224
_TB = 8
_ROWS = _TB * _MP


def _build_masks():
    rows = []
    for n in range(1, 6):
        for combo in combinations(range(_NCARDS), n):
            m = np.zeros(_NCARDS, dtype=np.float32)
            m[list(combo)] = 1.0
            rows.append(m)
    return np.stack(rows, axis=0)


_MASKS = np.zeros((_MP, _NCARDS), dtype=np.float32)
_MASKS[:_M] = _build_masks()

_A_IN = np.zeros((_ROWS, _TB * _NCARDS), dtype=np.float32)
_A_OUT = np.zeros((_ROWS, _TB * _NCARDS), dtype=np.float32)
_A_B = np.zeros((_ROWS, _TB), dtype=np.float32)
for _b in range(_TB):
    _A_IN[_b * _MP:(_b + 1) * _MP, _b * _NCARDS:(_b + 1) * _NCARDS] = _MASKS
    _A_OUT[_b * _MP:(_b + 1) * _MP, _b * _NCARDS:(_b + 1) * _NCARDS] = 1.0 - _MASKS
    _A_B[_b * _MP:(_b + 1) * _MP, _b] = 1.0

_SIZES2 = np.zeros((_ROWS, 2), dtype=np.float32)
_SIZES2[:, 0] = np.tile(_MASKS.sum(1), _TB)
_SIZES2[:, 1] = _NCARDS - _SIZES2[:, 0]

_STRAIGHT = np.zeros((16, 14), dtype=np.float32)
_STRAIGHT[:10] = np.array([
    [0,1,1,1,1,1,0,0,0,0,0,0,0,0],
    [0,0,1,1,1,1,1,0,0,0,0,0,0,0],
    [0,0,0,1,1,1,1,1,0,0,0,0,0,0],
    [0,0,0,0,1,1,1,1,1,0,0,0,0,0],
    [0,0,0,0,0,1,1,1,1,1,0,0,0,0],
    [0,0,0,0,0,0,1,1,1,1,1,0,0,0],
    [0,0,0,0,0,0,0,1,1,1,1,1,0,0],
    [0,0,0,0,0,0,0,0,1,1,1,1,1,0],
    [0,0,0,0,0,0,0,0,0,1,1,1,1,1],
    [0,1,1,1,1,0,0,0,0,0,0,0,0,1]], dtype=np.float32)


def _gelu(x):
    return x * 0.5 * (1.0 + jax.lax.erf(x * np.float32(1.0 / np.sqrt(2.0))))


def _dot(a, b):
    return jnp.dot(a, b, preferred_element_type=jnp.float32)


def _body(cards_ref, flat_ref, ain_ref, aout_ref, ab_ref, sizes_ref,
          str_ref, w0f_ref, w0md_ref, b0_ref, w1_ref, b1_ref, w2_ref, b2_ref,
          w3_ref, b3_ref, a0f_ref, a0md_ref, a0b_ref, a1_ref, a1b_ref,
          a2_ref, a2b_ref, log_ref, aux_ref):
    f32 = jnp.float32
    cards = cards_ref[...]
    rank = cards[:, 0:1]
    suit = cards[:, 1:2]
    enh = cards[:, 2:3]

    iota14 = jax.lax.broadcasted_iota(jnp.int32, (1, 14), 1)
    iota6 = jax.lax.broadcasted_iota(jnp.int32, (1, 6), 1)

    oh_r = ((rank == iota14) & (iota14 != 0)).astype(f32)
    oh_s = ((suit == iota6) & (iota6 != 0)).astype(f32)
    inv = ((rank == 0) & (suit == 0) & (enh == 0)).astype(f32)
    hits_pre = jax.lax.dot_general(oh_r, str_ref[...], (((1,), (1,)), ((), ())),
                                   preferred_element_type=f32)
    x_cards = jnp.concatenate(
        [oh_r, oh_s, inv, jnp.zeros((_TB * _NCARDS, 3), f32), hits_pre],
        axis=-1)

    s_in = _dot(ain_ref[...], x_cards)
    s_out = _dot(aout_ref[...], x_cards)
    sizes2 = sizes_ref[...]
    invc = s_in[:, 20:21]

    def metadata(s, sizes):
        rc = s[:, 0:14]
        sc = s[:, 14:20]
        hits = s[:, 24:40]
        msr = rc.max(axis=-1, keepdims=True)
        distinct = (rc > 0).astype(f32).sum(axis=-1, keepdims=True)
        mss = sc.max(axis=-1, keepdims=True) + sc[:, 5:6]
        msh = hits.max(axis=-1, keepdims=True)
        is_straight = (sizes >= 5) & (msh == 5) & (distinct >= 5)
        is_flush = (sizes >= 5) & (mss >= 5)
        is_sf = is_straight & is_flush
        has_pair = (sizes >= 2) & (msr >= 2)
        has_three = (sizes >= 3) & (msr >= 3)
        has_four = (sizes >= 4) & (msr >= 4)
        has_fh = (sizes >= 5) & has_three & (~has_four) & (distinct == 2)
        num_pairs = (rc >= 2).astype(f32).sum(axis=-1, keepdims=True)
        has_two_pair = (sizes >= 4) & (num_pairs == 2)
        return jnp.concatenate(
            [b.astype(f32) for b in
             (has_pair, has_three, has_four, has_fh, has_two_pair,
              is_straight, is_flush, is_sf)]
            + [msr, mss, distinct, msh, sizes, rc, sc], axis=-1)

    md = jnp.concatenate(
        [metadata(s_in, sizes2[:, 0:1]),
         metadata(s_out, sizes2[:, 1:2]),
         jnp.zeros((_ROWS, 6), f32)], axis=-1)

    flat = flat_ref[...]
    ab = ab_ref[...]
    yh = _dot(ab, _dot(flat, w0f_ref[...]))
    yg = _dot(ab, _dot(flat, a0f_ref[...]))

    h0 = _gelu(_dot(md, w0md_ref[...]) + yh + b0_ref[...])
    h1 = _gelu(_dot(h0, w1_ref[...]) + b1_ref[...])
    h2 = _gelu(_dot(h1, w2_ref[...]) + b2_ref[...])
    logit = (h2 * w3_ref[...]).sum(axis=-1, keepdims=True) + b3_ref[...]

    g0 = _gelu(_dot(md, a0md_ref[...]) + yg + a0b_ref[...])
    g1 = _gelu(_dot(g0, a1_ref[...]) + a1b_ref[...])
    aux = (g1 * a2_ref[...]).sum(axis=-1, keepdims=True) + a2b_ref[...]

    valid = invc == 0.0
    log_ref[...] = jnp.where(valid, logit, -1e9)
    aux_ref[...] = jnp.where(valid, aux, 0.0)


def kernel(embeds, flat_info, cards_rank, cards_suit, cards_enhancement,
           W0, b0, W1, b1, W2, b2, W3, b3, A0, a0, A1, a1, A2, a2):
    del embeds
    B = flat_info.shape[0]
    f32 = jnp.float32
    cards = jnp.stack(
        [cards_rank.astype(jnp.int32), cards_suit.astype(jnp.int32),
         cards_enhancement.astype(jnp.int32)], axis=-1).reshape(B * _NCARDS, 3)

    flat_w = int(flat_info.shape[-1])
    W0f, W0md_raw = W0[:flat_w], W0[flat_w:]
    A0f, A0md_raw = A0[:flat_w], A0[flat_w:]
    W0md = jnp.concatenate([W0md_raw, jnp.zeros((6, W0.shape[1]), f32)], axis=0)
    A0md = jnp.concatenate([A0md_raw, jnp.zeros((6, A0.shape[1]), f32)], axis=0)

    grid = B // _TB

    def tiled(shape):
        return pl.BlockSpec(shape, lambda i: (i, 0))

    def whole(x):
        return pl.BlockSpec(x.shape, lambda i: (0,) * x.ndim)

    operands = (
        cards, flat_info,
        jnp.asarray(_A_IN), jnp.asarray(_A_OUT), jnp.asarray(_A_B),
        jnp.asarray(_SIZES2), jnp.asarray(_STRAIGHT),
        W0f, W0md, b0[None], W1, b1[None], W2, b2[None],
        W3.T, b3[None], A0f, A0md, a0[None], A1, a1[None], A2.T, a2[None],
    )
    in_specs = [
        tiled((_TB * _NCARDS, 3)), tiled((_TB, flat_w)),
    ] + [whole(x) for x in operands[2:]]

    out_shape = [jax.ShapeDtypeStruct((B * _MP, 1), f32)] * 2
    out_specs = [pl.BlockSpec((_ROWS, 1), lambda i: (i, 0))] * 2

    logits_col, aux_col = pl.pallas_call(
        _body,
        grid=(grid,),
        in_specs=in_specs,
        out_specs=out_specs,
        out_shape=out_shape,
    )(*operands)

    logits = logits_col.reshape(B, _MP)[:, :_M]
    aux = aux_col.reshape(B, _MP)[:, :_M, None]
    return logits, aux


# speedup vs baseline: 24.2122x; 2.4013x over previous
from itertools import combinations

import numpy as np
import jax
import jax.numpy as jnp
from jax.experimental import pallas as pl

_NCARDS = 8
_M = 218
_MP = 224
_TB = 8
_ROWS = _TB * _MP


def _build_masks():
    rows = []
    for n in range(1, 6):
        for combo in combinations(range(_NCARDS), n):
            m = np.zeros(_NCARDS, dtype=np.float32)
            m[list(combo)] = 1.0
            rows.append(m)
    return np.stack(rows, axis=0)


_MASKS = np.zeros((_MP, _NCARDS), dtype=np.float32)
_MASKS[:_M] = _build_masks()

_A_CAT = np.zeros((_ROWS, 2 * _TB * _NCARDS), dtype=np.float32)
_A_B = np.zeros((_ROWS, _TB), dtype=np.float32)
for _b in range(_TB):
    _r0, _c0 = _b * _MP, _b * _NCARDS
    _A_CAT[_r0:_r0 + _MP, _c0:_c0 + _NCARDS] = _MASKS
    _A_CAT[_r0:_r0 + _MP, 64 + _c0:64 + _c0 + _NCARDS] = 1.0 - _MASKS
    _A_B[_r0:_r0 + _MP, _b] = 1.0

_SIZES2 = np.zeros((_ROWS, 2), dtype=np.float32)
_SIZES2[:, 0] = np.tile(_MASKS.sum(1), _TB)
_SIZES2[:, 1] = _NCARDS - _SIZES2[:, 0]

_STRAIGHT = np.zeros((16, 14), dtype=np.float32)
_STRAIGHT[:10] = np.array([
    [0,1,1,1,1,1,0,0,0,0,0,0,0,0],
    [0,0,1,1,1,1,1,0,0,0,0,0,0,0],
    [0,0,0,1,1,1,1,1,0,0,0,0,0,0],
    [0,0,0,0,1,1,1,1,1,0,0,0,0,0],
    [0,0,0,0,0,1,1,1,1,1,0,0,0,0],
    [0,0,0,0,0,0,1,1,1,1,1,0,0,0],
    [0,0,0,0,0,0,0,1,1,1,1,1,0,0],
    [0,0,0,0,0,0,0,0,1,1,1,1,1,0],
    [0,0,0,0,0,0,0,0,0,1,1,1,1,1],
    [0,1,1,1,1,0,0,0,0,0,0,0,0,1]], dtype=np.float32)

_RC_IN, _SC_IN = range(0, 14), range(14, 20)
_RC_OUT, _SC_OUT = range(20, 34), range(34, 40)
_HI_IN, _HI_OUT = range(40, 56), range(56, 72)
_SC5_IN, _SC5_OUT = 19, 39

_SEG = np.zeros((80, 32), dtype=np.float32)
_NSEG = np.zeros((1, 32), dtype=np.float32)
_S1 = np.zeros((80, 32), dtype=np.float32)
_S2 = np.zeros((80, 32), dtype=np.float32)
_C4 = np.zeros((80, 32), dtype=np.float32)


def _seg_col(col, lanes):
    for l in lanes:
        _SEG[l, col] = 1.0
    _NSEG[0, col] = len(lanes)


for _half, (_rc, _sc, _hi, _sc5) in enumerate(
        [(_RC_IN, _SC_IN, _HI_IN, _SC5_IN),
         (_RC_OUT, _SC_OUT, _HI_OUT, _SC5_OUT)]):
    _o = 8 * _half
    for _c in (0, 1, 2):
        _seg_col(_o + _c, _rc)
    _seg_col(_o + 3, _sc)
    _C4[_sc5, _o + 3] = 1.0
    _seg_col(_o + 4, _hi)
    for _c in (5, 6):
        for _l in _rc:
            _S1[_l, _o + _c] = 1.0
    for _l in _rc:
        _S2[_l, _o + 7] = 1.0
    _ro = 16 + 4 * _half
    _seg_col(_ro + 0, _rc)
    _seg_col(_ro + 1, _sc)
    _C4[_sc5, _ro + 1] = 1.0
    for _l in _rc:
        _S1[_l, _ro + 2] = 1.0
    _seg_col(_ro + 3, _hi)

_TLO = np.array([[2, 3, 4, 5, 5, 5, 2, 2] * 2], dtype=np.float32)
_THI = np.array([[1e9, 1e9, 1e9, 1e9, 5, 1e9, 2, 2] * 2], dtype=np.float32)

_CF = np.zeros((16, 16), dtype=np.float32)
_FLAG_BASE = [1, 1, 1, 2, 1, 2, 1, 3]
_FLAG_SZREQ = [2, 3, 4, 5, 4, 5, 5, 5]
for _half in range(2):
    _o = 8 * _half
    _CF[_o + 0, _o + 0] = 1.0
    _CF[_o + 1, _o + 1] = 1.0
    _CF[_o + 2, _o + 2] = 1.0
    _CF[_o + 1, _o + 3] = 1.0
    _CF[_o + 2, _o + 3] = -1.0
    _CF[_o + 6, _o + 3] = 1.0
    _CF[_o + 7, _o + 4] = 1.0
    _CF[_o + 4, _o + 5] = 1.0
    _CF[_o + 5, _o + 5] = 1.0
    _CF[_o + 3, _o + 6] = 1.0
    _CF[_o + 4, _o + 7] = 1.0
    _CF[_o + 5, _o + 7] = 1.0
    _CF[_o + 3, _o + 7] = 1.0

_NREQ = np.zeros((_ROWS, 16), dtype=np.float32)
for _j in range(16):
    _sz = _SIZES2[:, _j // 8]
    _NREQ[:, _j] = np.where(_sz >= _FLAG_SZREQ[_j % 8],
                            _FLAG_BASE[_j % 8], 99.0)

_PERM = (list(range(13, 33)) + list(range(46, 66)) +
         list(range(0, 8)) + list(range(33, 41)) +
         [8, 9, 10, 11, 41, 42, 43, 44, 12, 45])


def _gelu(x):
    return x * 0.5 * (1.0 + jax.lax.erf(x * np.float32(1.0 / np.sqrt(2.0))))


def _dot(a, b):
    return jnp.dot(a, b, preferred_element_type=jnp.float32)


def _body(cards_ref, flat_ref, acat_ref, ab_ref, sizes_ref, nreq_ref,
          str_ref, seg_ref, nseg_ref, s1_ref, s2_ref, c4_ref, tlo_ref,
          thi_ref, cf_ref, w0f2_ref, b0a0_ref, w01_ref, w1a1_ref, b1a1_ref,
          cat2_ref, b2_ref, a2b_ref, w3_ref, b3_ref, log_ref, aux_ref):
    f32 = jnp.float32
    cards = cards_ref[...]
    rank = cards[:, 0:1]
    suit = cards[:, 1:2]
    enh = cards[:, 2:3]

    iota14 = jax.lax.broadcasted_iota(jnp.int32, (1, 14), 1)
    iota6 = jax.lax.broadcasted_iota(jnp.int32, (1, 6), 1)
    oh_r = ((rank == iota14) & (iota14 != 0)).astype(f32)
    oh_s = ((suit == iota6) & (iota6 != 0)).astype(f32)
    inv = ((rank == 0) & (suit == 0) & (enh == 0)).astype(f32)
    hits_pre = jax.lax.dot_general(oh_r, str_ref[...], (((1,), (1,)), ((), ())),
                                   preferred_element_type=f32)

    nc = _TB * _NCARDS

    def z(n):
        return jnp.zeros((nc, n), f32)

    row_in = jnp.concatenate(
        [oh_r, oh_s, z(20), hits_pre, z(16), inv, z(7)], axis=-1)
    row_out = jnp.concatenate(
        [z(20), oh_r, oh_s, z(16), hits_pre, z(8)], axis=-1)
    x2 = jnp.concatenate([row_in, row_out], axis=0)

    s_full = _dot(acat_ref[...], x2)

    e = jnp.exp2(4.0 * s_full)
    p_a = jnp.floor(
        jnp.log2(jnp.maximum(_dot(e, seg_ref[...]) - nseg_ref[...], 1.0))
        * 0.25 + 0.03)
    ipos = (s_full > 0).astype(f32)
    ige2 = (s_full >= 2).astype(f32)
    p = (p_a + _dot(ipos, s1_ref[...]) + _dot(ige2, s2_ref[...])
         + _dot(s_full, c4_ref[...]))

    v = ((p[:, 0:16] >= tlo_ref[...]) &
         (p[:, 0:16] <= thi_ref[...])).astype(f32)
    flags = (_dot(v, cf_ref[...]) == nreq_ref[...]).astype(f32)

    md = jnp.concatenate(
        [s_full[:, 0:40], flags, p[:, 16:24], sizes_ref[...],
         jnp.zeros((_ROWS, 6), f32)], axis=-1)

    flat = flat_ref[...]
    yhg = _dot(ab_ref[...], _dot(flat, w0f2_ref[...]) + b0a0_ref[...])

    hg0 = _gelu(_dot(md, w01_ref[...]) + yhg)
    hg1 = _gelu(_dot(hg0, w1a1_ref[...]) + b1a1_ref[...])
    t2 = _dot(hg1, cat2_ref[...])
    h2 = _gelu(t2[:, 0:64] + b2_ref[...])
    aux = t2[:, 64:65] + a2b_ref[...]
    logit = _dot(h2, w3_ref[...])[:, 0:1] + b3_ref[...]

    valid = s_full[:, 72:73] == 0.0
    log_ref[...] = jnp.where(valid, logit, -1e9)
    aux_ref[...] = jnp.where(valid, aux, 0.0)


def kernel(embeds, flat_info, cards_rank, cards_suit, cards_enhancement,
           W0, b0, W1, b1, W2, b2, W3, b3, A0, a0, A1, a1, A2, a2):
    del embeds
    B = flat_info.shape[0]
    f32 = jnp.float32
    cards = jnp.stack(
        [cards_rank.astype(jnp.int32), cards_suit.astype(jnp.int32),
         cards_enhancement.astype(jnp.int32)], axis=-1).reshape(B * _NCARDS, 3)

    flat_w = int(flat_info.shape[-1])
    perm = jnp.asarray(np.asarray(_PERM, np.int32))
    pad6 = jnp.zeros((6, 256), f32)
    w0md = jnp.concatenate([W0[flat_w:][perm], pad6], axis=0)
    a0md = jnp.concatenate([A0[flat_w:][perm], pad6], axis=0)
    w01 = jnp.concatenate([w0md, a0md], axis=1)
    w0f2 = jnp.concatenate([W0[:flat_w], A0[:flat_w]], axis=1)
    b0a0 = jnp.concatenate([b0, a0])[None]
    w1a1 = jnp.zeros((512, 256), f32)
    w1a1 = w1a1.at[0:256, 0:128].set(W1).at[256:512, 128:256].set(A1)
    b1a1 = jnp.concatenate([b1, a1])[None]
    cat2 = jnp.zeros((256, 72), f32)
    cat2 = cat2.at[0:128, 0:64].set(W2).at[128:256, 64:65].set(A2)
    w3p = jnp.zeros((64, 8), f32).at[:, 0:1].set(W3)

    grid = B // _TB

    def tiled(shape):
        return pl.BlockSpec(shape, lambda i: (i, 0))

    def whole(x):
        return pl.BlockSpec(x.shape, lambda i: (0,) * x.ndim)

    operands = (
        cards, flat_info,
        jnp.asarray(_A_CAT), jnp.asarray(_A_B), jnp.asarray(_SIZES2),
        jnp.asarray(_NREQ), jnp.asarray(_STRAIGHT), jnp.asarray(_SEG),
        jnp.asarray(_NSEG), jnp.asarray(_S1), jnp.asarray(_S2),
        jnp.asarray(_C4), jnp.asarray(_TLO), jnp.asarray(_THI),
        jnp.asarray(_CF),
        w0f2, b0a0, w01, w1a1, b1a1, cat2, b2[None], a2[None], w3p, b3[None],
    )
    in_specs = [
        tiled((_TB * _NCARDS, 3)), tiled((_TB, flat_w)),
    ] + [whole(x) for x in operands[2:]]

    out_shape = [jax.ShapeDtypeStruct((B * _MP, 1), f32)] * 2
    out_specs = [pl.BlockSpec((_ROWS, 1), lambda i: (i, 0))] * 2

    logits_col, aux_col = pl.pallas_call(
        _body,
        grid=(grid,),
        in_specs=in_specs,
        out_specs=out_specs,
        out_shape=out_shape,
    )(*operands)

    logits = logits_col.reshape(B, _MP)[:, :_M]
    aux = aux_col.reshape(B, _MP)[:, :_M, None]
    return logits, aux


# speedup vs baseline: 28.3360x; 1.1703x over previous
from itertools import combinations

import numpy as np
import jax
import jax.numpy as jnp
from jax.experimental import pallas as pl

_NCARDS = 8
_M = 218
_MP = 224
_TB = 8
_ROWS = _TB * _MP


def _build_masks():
    rows = []
    for n in range(1, 6):
        for combo in combinations(range(_NCARDS), n):
            m = np.zeros(_NCARDS, dtype=np.float32)
            m[list(combo)] = 1.0
            rows.append(m)
    return np.stack(rows, axis=0)


_MASKS = np.zeros((_MP, _NCARDS), dtype=np.float32)
_MASKS[:_M] = _build_masks()

_A_CAT = np.zeros((_ROWS, 2 * _TB * _NCARDS), dtype=np.float32)
_A_B = np.zeros((_ROWS, _TB), dtype=np.float32)
for _b in range(_TB):
    _r0, _c0 = _b * _MP, _b * _NCARDS
    _A_CAT[_r0:_r0 + _MP, _c0:_c0 + _NCARDS] = _MASKS
    _A_CAT[_r0:_r0 + _MP, 64 + _c0:64 + _c0 + _NCARDS] = 1.0 - _MASKS
    _A_B[_r0:_r0 + _MP, _b] = 1.0

_SIZES2 = np.zeros((_ROWS, 2), dtype=np.float32)
_SIZES2[:, 0] = np.tile(_MASKS.sum(1), _TB)
_SIZES2[:, 1] = _NCARDS - _SIZES2[:, 0]

_STRAIGHT = np.zeros((16, 14), dtype=np.float32)
_STRAIGHT[:10] = np.array([
    [0,1,1,1,1,1,0,0,0,0,0,0,0,0],
    [0,0,1,1,1,1,1,0,0,0,0,0,0,0],
    [0,0,0,1,1,1,1,1,0,0,0,0,0,0],
    [0,0,0,0,1,1,1,1,1,0,0,0,0,0],
    [0,0,0,0,0,1,1,1,1,1,0,0,0,0],
    [0,0,0,0,0,0,1,1,1,1,1,0,0,0],
    [0,0,0,0,0,0,0,1,1,1,1,1,0,0],
    [0,0,0,0,0,0,0,0,1,1,1,1,1,0],
    [0,0,0,0,0,0,0,0,0,1,1,1,1,1],
    [0,1,1,1,1,0,0,0,0,0,0,0,0,1]], dtype=np.float32)

_RC_IN, _SC_IN = range(0, 14), range(14, 20)
_RC_OUT, _SC_OUT = range(20, 34), range(34, 40)
_HI_IN, _HI_OUT = range(40, 56), range(56, 72)
_SC5_IN, _SC5_OUT = 19, 39

_SEG = np.zeros((128, 32), dtype=np.float32)
_NSEG = np.zeros((1, 32), dtype=np.float32)
_PM = np.zeros((384, 32), dtype=np.float32)


def _seg_col(col, lanes):
    for l in lanes:
        _SEG[l, col] = 1.0
    _NSEG[0, col] = len(lanes)


for _half, (_rc, _sc, _hi, _sc5) in enumerate(
        [(_RC_IN, _SC_IN, _HI_IN, _SC5_IN),
         (_RC_OUT, _SC_OUT, _HI_OUT, _SC5_OUT)]):
    _o = 8 * _half
    for _c in (0, 1, 2):
        _seg_col(_o + _c, _rc)
    _seg_col(_o + 3, _sc)
    _PM[_sc5, _o + 3] = 1.0
    _seg_col(_o + 4, _hi)
    for _c in (5, 6):
        for _l in _rc:
            _PM[128 + _l, _o + _c] = 1.0
    for _l in _rc:
        _PM[256 + _l, _o + 7] = 1.0
    _ro = 16 + 4 * _half
    _seg_col(_ro + 0, _rc)
    _seg_col(_ro + 1, _sc)
    _PM[_sc5, _ro + 1] = 1.0
    for _l in _rc:
        _PM[128 + _l, _ro + 2] = 1.0
    _seg_col(_ro + 3, _hi)

_TLO = np.array([[2, 3, 4, 5, 5, 5, 2, 2] * 2], dtype=np.float32)
_THI = np.array([[1e9, 1e9, 1e9, 1e9, 5, 1e9, 2, 2] * 2], dtype=np.float32)

_CF = np.zeros((16, 16), dtype=np.float32)
_FLAG_BASE = [1, 1, 1, 2, 1, 2, 1, 3]
_FLAG_SZREQ = [2, 3, 4, 5, 4, 5, 5, 5]
for _half in range(2):
    _o = 8 * _half
    _CF[_o + 0, _o + 0] = 1.0
    _CF[_o + 1, _o + 1] = 1.0
    _CF[_o + 2, _o + 2] = 1.0
    _CF[_o + 1, _o + 3] = 1.0
    _CF[_o + 2, _o + 3] = -1.0
    _CF[_o + 6, _o + 3] = 1.0
    _CF[_o + 7, _o + 4] = 1.0
    _CF[_o + 4, _o + 5] = 1.0
    _CF[_o + 5, _o + 5] = 1.0
    _CF[_o + 3, _o + 6] = 1.0
    _CF[_o + 4, _o + 7] = 1.0
    _CF[_o + 5, _o + 7] = 1.0
    _CF[_o + 3, _o + 7] = 1.0

_NREQ = np.zeros((_ROWS, 16), dtype=np.float32)
for _j in range(16):
    _sz = _SIZES2[:, _j // 8]
    _NREQ[:, _j] = np.where(_sz >= _FLAG_SZREQ[_j % 8],
                            _FLAG_BASE[_j % 8], 99.0)

_PERM = (list(range(13, 33)) + list(range(46, 66)) +
         list(range(0, 8)) + list(range(33, 41)) +
         [8, 9, 10, 11, 41, 42, 43, 44, 12, 45])

_RSQRT2 = np.float32(1.0 / np.sqrt(2.0))
_HSQRT2 = np.float32(np.sqrt(2.0) / 2.0)


def _gelu_pre(t):
    return (t * _HSQRT2) * (1.0 + jax.lax.erf(t))


def _dot(a, b):
    return jnp.dot(a, b, preferred_element_type=jnp.float32)


def _body(cards_ref, flat_ref, acat_ref, ab_ref, sizes_ref, nreq_ref,
          str_ref, seg_ref, nseg_ref, pm_ref, tlo_ref, thi_ref, cf_ref,
          w0f2_ref, b0a0_ref, w01_ref, w1a1_ref, b1a1_ref,
          cat2_ref, b2_ref, a2b_ref, w3_ref, b3_ref, log_ref, aux_ref):
    f32 = jnp.float32
    bf16 = jnp.bfloat16
    cards = cards_ref[...]
    rank = cards[:, 0:1]
    suit = cards[:, 1:2]
    enh = cards[:, 2:3]

    iota14 = jax.lax.broadcasted_iota(jnp.int32, (1, 14), 1)
    iota6 = jax.lax.broadcasted_iota(jnp.int32, (1, 6), 1)
    oh_r = ((rank == iota14) & (iota14 != 0)).astype(bf16)
    oh_s = ((suit == iota6) & (iota6 != 0)).astype(bf16)
    inv = ((rank == 0) & (suit == 0) & (enh == 0)).astype(bf16)
    hits_pre = jax.lax.dot_general(
        oh_r, str_ref[...], (((1,), (1,)), ((), ())),
        preferred_element_type=jnp.float32).astype(bf16)

    nc = _TB * _NCARDS

    def z(n):
        return jnp.zeros((nc, n), bf16)

    row_in = jnp.concatenate(
        [oh_r, oh_s, z(20), hits_pre, z(16), inv, z(55)], axis=-1)
    row_out = jnp.concatenate(
        [z(20), oh_r, oh_s, z(16), hits_pre, z(56)], axis=-1)
    x2 = jnp.concatenate([row_in, row_out], axis=0)

    s_full = _dot(acat_ref[...], x2)

    e = jnp.exp2(4.0 * s_full).astype(bf16)
    p_a = jnp.floor(
        jnp.log2(jnp.maximum(_dot(e, seg_ref[...]) - nseg_ref[...], 1.0))
        * 0.25 + 0.03)
    sb = s_full.astype(bf16)
    q = jnp.concatenate(
        [sb, (s_full > 0).astype(bf16), (s_full >= 2).astype(bf16)],
        axis=-1)
    p = p_a + _dot(q, pm_ref[...])

    v = ((p[:, 0:16] >= tlo_ref[...]) &
         (p[:, 0:16] <= thi_ref[...])).astype(bf16)
    flags = (_dot(v, cf_ref[...]) == nreq_ref[...]).astype(bf16)

    md = jnp.concatenate(
        [sb[:, 0:40], flags, p[:, 16:24].astype(bf16), sizes_ref[...],
         ab_ref[...]], axis=-1)

    flat = flat_ref[...]
    w0dyn = (_dot(flat, w0f2_ref[...]) + b0a0_ref[...]).astype(bf16)
    w01_full = jnp.concatenate([w01_ref[...], w0dyn], axis=0)

    hg0 = _gelu_pre(_dot(md, w01_full)).astype(bf16)
    hg1 = _gelu_pre(_dot(hg0, w1a1_ref[...]) + b1a1_ref[...]).astype(bf16)
    t2 = _dot(hg1, cat2_ref[...])
    h2 = _gelu_pre(t2[:, 0:64] + b2_ref[...]).astype(bf16)
    aux = t2[:, 64:65] + a2b_ref[...]
    logit = _dot(h2, w3_ref[...])[:, 0:1] + b3_ref[...]

    valid = s_full[:, 72:73] == 0.0
    log_ref[...] = jnp.where(valid, logit, -1e9)
    aux_ref[...] = jnp.where(valid, aux, 0.0)


def kernel(embeds, flat_info, cards_rank, cards_suit, cards_enhancement,
           W0, b0, W1, b1, W2, b2, W3, b3, A0, a0, A1, a1, A2, a2):
    del embeds
    B = flat_info.shape[0]
    f32 = jnp.float32
    bf16 = jnp.bfloat16
    cards = jnp.stack(
        [cards_rank.astype(jnp.int32), cards_suit.astype(jnp.int32),
         cards_enhancement.astype(jnp.int32)], axis=-1).reshape(B * _NCARDS, 3)

    flat_w = int(flat_info.shape[-1])
    perm = jnp.asarray(np.asarray(_PERM, np.int32))
    r2 = _RSQRT2
    w0md = W0[flat_w:][perm] * r2
    a0md = A0[flat_w:][perm] * r2
    w01 = jnp.concatenate([w0md, a0md], axis=1).astype(bf16)
    w0f2 = (jnp.concatenate([W0[:flat_w], A0[:flat_w]], axis=1)
            * r2).astype(bf16)
    b0a0 = (jnp.concatenate([b0, a0]) * r2)[None]
    w1a1 = jnp.zeros((512, 256), f32)
    w1a1 = (w1a1.at[0:256, 0:128].set(W1)
            .at[256:512, 128:256].set(A1) * r2).astype(bf16)
    b1a1 = (jnp.concatenate([b1, a1]) * r2)[None]
    cat2 = jnp.zeros((256, 72), f32)
    cat2 = cat2.at[0:128, 0:64].set(W2 * r2).at[128:256, 64:65].set(A2)
    cat2 = cat2.astype(bf16)
    w3p = jnp.zeros((64, 8), f32).at[:, 0:1].set(W3).astype(bf16)

    grid = B // _TB

    def tiled(shape):
        return pl.BlockSpec(shape, lambda i: (i, 0))

    def whole(x):
        return pl.BlockSpec(x.shape, lambda i: (0,) * x.ndim)

    operands = (
        cards, flat_info.astype(bf16),
        jnp.asarray(_A_CAT, bf16), jnp.asarray(_A_B, bf16),
        jnp.asarray(_SIZES2, bf16), jnp.asarray(_NREQ),
        jnp.asarray(_STRAIGHT, bf16), jnp.asarray(_SEG, bf16),
        jnp.asarray(_NSEG), jnp.asarray(_PM, bf16),
        jnp.asarray(_TLO), jnp.asarray(_THI), jnp.asarray(_CF, bf16),
        w0f2, b0a0, w01, w1a1, b1a1, cat2,
        (b2 * r2)[None], a2[None], w3p, b3[None],
    )
    in_specs = [
        tiled((_TB * _NCARDS, 3)), tiled((_TB, flat_w)),
    ] + [whole(x) for x in operands[2:]]

    out_shape = [jax.ShapeDtypeStruct((B * _MP, 1), f32)] * 2
    out_specs = [pl.BlockSpec((_ROWS, 1), lambda i: (i, 0))] * 2

    logits_col, aux_col = pl.pallas_call(
        _body,
        grid=(grid,),
        in_specs=in_specs,
        out_specs=out_specs,
        out_shape=out_shape,
    )(*operands)

    logits = logits_col.reshape(B, _MP)[:, :_M]
    aux = aux_col.reshape(B, _MP)[:, :_M, None]
    return logits, aux


# speedup vs baseline: 28.8857x; 1.0194x over previous
from itertools import combinations

import numpy as np
import jax
import jax.numpy as jnp
from jax.experimental import pallas as pl

_NCARDS = 8
_M = 218
_MP = 224
_TB = 8
_ROWS = _TB * _MP


def _build_masks():
    rows = []
    for n in range(1, 6):
        for combo in combinations(range(_NCARDS), n):
            m = np.zeros(_NCARDS, dtype=np.float32)
            m[list(combo)] = 1.0
            rows.append(m)
    return np.stack(rows, axis=0)


_MASKS = np.zeros((_MP, _NCARDS), dtype=np.float32)
_MASKS[:_M] = _build_masks()

_A_CAT = np.zeros((_ROWS, 2 * _TB * _NCARDS), dtype=np.float32)
_A_B = np.zeros((_ROWS, _TB), dtype=np.float32)
for _b in range(_TB):
    _r0, _c0 = _b * _MP, _b * _NCARDS
    _A_CAT[_r0:_r0 + _MP, _c0:_c0 + _NCARDS] = _MASKS
    _A_CAT[_r0:_r0 + _MP, 64 + _c0:64 + _c0 + _NCARDS] = 1.0 - _MASKS
    _A_B[_r0:_r0 + _MP, _b] = 1.0

_SIZES2 = np.zeros((_ROWS, 2), dtype=np.float32)
_SIZES2[:, 0] = np.tile(_MASKS.sum(1), _TB)
_SIZES2[:, 1] = _NCARDS - _SIZES2[:, 0]

_STRAIGHT = np.zeros((16, 14), dtype=np.float32)
_STRAIGHT[:10] = np.array([
    [0,1,1,1,1,1,0,0,0,0,0,0,0,0],
    [0,0,1,1,1,1,1,0,0,0,0,0,0,0],
    [0,0,0,1,1,1,1,1,0,0,0,0,0,0],
    [0,0,0,0,1,1,1,1,1,0,0,0,0,0],
    [0,0,0,0,0,1,1,1,1,1,0,0,0,0],
    [0,0,0,0,0,0,1,1,1,1,1,0,0,0],
    [0,0,0,0,0,0,0,1,1,1,1,1,0,0],
    [0,0,0,0,0,0,0,0,1,1,1,1,1,0],
    [0,0,0,0,0,0,0,0,0,1,1,1,1,1],
    [0,1,1,1,1,0,0,0,0,0,0,0,0,1]], dtype=np.float32)

_RC_IN, _SC_IN = range(0, 14), range(14, 20)
_RC_OUT, _SC_OUT = range(20, 34), range(34, 40)
_HI_IN, _HI_OUT = range(40, 56), range(56, 72)
_SC5_IN, _SC5_OUT = 19, 39

_SEG = np.zeros((128, 32), dtype=np.float32)
_NSEG = np.zeros((1, 32), dtype=np.float32)
_PM = np.zeros((384, 32), dtype=np.float32)


def _seg_col(col, lanes):
    for l in lanes:
        _SEG[l, col] = 1.0
    _NSEG[0, col] = len(lanes)


for _half, (_rc, _sc, _hi, _sc5) in enumerate(
        [(_RC_IN, _SC_IN, _HI_IN, _SC5_IN),
         (_RC_OUT, _SC_OUT, _HI_OUT, _SC5_OUT)]):
    _o = 8 * _half
    for _c in (0, 1, 2):
        _seg_col(_o + _c, _rc)
    _seg_col(_o + 3, _sc)
    _PM[_sc5, _o + 3] = 1.0
    _seg_col(_o + 4, _hi)
    for _c in (5, 6):
        for _l in _rc:
            _PM[128 + _l, _o + _c] = 1.0
    for _l in _rc:
        _PM[256 + _l, _o + 7] = 1.0
    _ro = 16 + 4 * _half
    _seg_col(_ro + 0, _rc)
    _seg_col(_ro + 1, _sc)
    _PM[_sc5, _ro + 1] = 1.0
    for _l in _rc:
        _PM[128 + _l, _ro + 2] = 1.0
    _seg_col(_ro + 3, _hi)

_TLO = np.array([[2, 3, 4, 5, 5, 5, 2, 2] * 2], dtype=np.float32)
_THI = np.array([[1e9, 1e9, 1e9, 1e9, 5, 1e9, 2, 2] * 2], dtype=np.float32)

_CF = np.zeros((16, 16), dtype=np.float32)
_FLAG_BASE = [1, 1, 1, 2, 1, 2, 1, 3]
_FLAG_SZREQ = [2, 3, 4, 5, 4, 5, 5, 5]
for _half in range(2):
    _o = 8 * _half
    _CF[_o + 0, _o + 0] = 1.0
    _CF[_o + 1, _o + 1] = 1.0
    _CF[_o + 2, _o + 2] = 1.0
    _CF[_o + 1, _o + 3] = 1.0
    _CF[_o + 2, _o + 3] = -1.0
    _CF[_o + 6, _o + 3] = 1.0
    _CF[_o + 7, _o + 4] = 1.0
    _CF[_o + 4, _o + 5] = 1.0
    _CF[_o + 5, _o + 5] = 1.0
    _CF[_o + 3, _o + 6] = 1.0
    _CF[_o + 4, _o + 7] = 1.0
    _CF[_o + 5, _o + 7] = 1.0
    _CF[_o + 3, _o + 7] = 1.0

_NREQ = np.zeros((_ROWS, 16), dtype=np.float32)
for _j in range(16):
    _sz = _SIZES2[:, _j // 8]
    _NREQ[:, _j] = np.where(_sz >= _FLAG_SZREQ[_j % 8],
                            _FLAG_BASE[_j % 8], 99.0)

_PERM = (list(range(13, 33)) + list(range(46, 66)) +
         list(range(0, 8)) + list(range(33, 41)) +
         [8, 9, 10, 11, 41, 42, 43, 44, 12, 45])

_RSQRT2 = np.float32(1.0 / np.sqrt(2.0))
_HSQRT2 = np.float32(np.sqrt(2.0) / 2.0)


def _gelu_pre(t):
    return (t * 0.7071067811865476) * (1.0 + jax.lax.erf(t))


def _dot(a, b):
    return jnp.dot(a, b, preferred_element_type=jnp.float32)


def _body(cards_ref, flat_ref, acat_ref, ab_ref, sizes_ref, nreq_ref,
          str_ref, seg_ref, nseg_ref, pm_ref, tlo_ref, thi_ref, cf_ref,
          w0f2_ref, b0a0_ref, w01_ref, w1a1_ref, b1a1_ref,
          cat2_ref, b2_ref, a2b_ref, w3_ref, b3_ref, log_ref, aux_ref):
    f32 = jnp.float32
    bf16 = jnp.bfloat16
    cards = cards_ref[...]
    rank = cards[:, 0:1]
    suit = cards[:, 1:2]
    enh = cards[:, 2:3]

    iota14 = jax.lax.broadcasted_iota(jnp.int32, (1, 14), 1)
    iota6 = jax.lax.broadcasted_iota(jnp.int32, (1, 6), 1)
    oh_r = ((rank == iota14) & (iota14 != 0)).astype(bf16)
    oh_s = ((suit == iota6) & (iota6 != 0)).astype(bf16)
    inv = ((rank == 0) & (suit == 0) & (enh == 0)).astype(bf16)
    hits_pre = jax.lax.dot_general(
        oh_r, str_ref[...], (((1,), (1,)), ((), ())),
        preferred_element_type=jnp.float32).astype(bf16)

    nc = _TB * _NCARDS

    def z(n):
        return jnp.zeros((nc, n), bf16)

    row_in = jnp.concatenate(
        [oh_r, oh_s, z(20), hits_pre, z(16), inv, z(55)], axis=-1)
    row_out = jnp.concatenate(
        [z(20), oh_r, oh_s, z(16), hits_pre, z(56)], axis=-1)
    x2 = jnp.concatenate([row_in, row_out], axis=0)

    s_full = _dot(acat_ref[...], x2)

    e = jnp.exp2(4.0 * s_full).astype(bf16)
    p_a = jnp.floor(
        jnp.log2(jnp.maximum(_dot(e, seg_ref[...]) - nseg_ref[...], 1.0))
        * 0.25 + 0.03)
    sb = s_full.astype(bf16)
    q = jnp.concatenate(
        [sb, (s_full > 0).astype(bf16), (s_full >= 2).astype(bf16)],
        axis=-1)
    p = p_a + _dot(q, pm_ref[...])

    v = ((p[:, 0:16] >= tlo_ref[...]) &
         (p[:, 0:16] <= thi_ref[...])).astype(bf16)
    flags = (_dot(v, cf_ref[...]) == nreq_ref[...]).astype(bf16)

    md = jnp.concatenate(
        [sb[:, 0:40], flags, p[:, 16:24].astype(bf16), sizes_ref[...],
         ab_ref[...]], axis=-1)

    flat = flat_ref[...]
    w0dyn = (_dot(flat, w0f2_ref[...]) + b0a0_ref[...]).astype(bf16)
    w01_full = jnp.concatenate([w01_ref[...], w0dyn], axis=0)

    hg0 = _gelu_pre(_dot(md, w01_full).astype(bf16))
    hg1 = _gelu_pre(
        (_dot(hg0, w1a1_ref[...]) + b1a1_ref[...]).astype(bf16))
    t2 = _dot(hg1, cat2_ref[...])
    h2 = _gelu_pre((t2[:, 0:64] + b2_ref[...]).astype(bf16))
    aux = t2[:, 64:65] + a2b_ref[...]
    logit = _dot(h2, w3_ref[...])[:, 0:1] + b3_ref[...]

    valid = s_full[:, 72:73] == 0.0
    log_ref[...] = jnp.where(valid, logit, -1e9)
    aux_ref[...] = jnp.where(valid, aux, 0.0)


def kernel(embeds, flat_info, cards_rank, cards_suit, cards_enhancement,
           W0, b0, W1, b1, W2, b2, W3, b3, A0, a0, A1, a1, A2, a2):
    del embeds
    B = flat_info.shape[0]
    f32 = jnp.float32
    bf16 = jnp.bfloat16
    cards = jnp.stack(
        [cards_rank.astype(jnp.int32), cards_suit.astype(jnp.int32),
         cards_enhancement.astype(jnp.int32)], axis=-1).reshape(B * _NCARDS, 3)

    flat_w = int(flat_info.shape[-1])
    perm = jnp.asarray(np.asarray(_PERM, np.int32))
    r2 = _RSQRT2
    w0md = W0[flat_w:][perm] * r2
    a0md = A0[flat_w:][perm] * r2
    w01 = jnp.concatenate([w0md, a0md], axis=1).astype(bf16)
    w0f2 = (jnp.concatenate([W0[:flat_w], A0[:flat_w]], axis=1)
            * r2).astype(bf16)
    b0a0 = (jnp.concatenate([b0, a0]) * r2)[None]
    w1a1 = jnp.zeros((512, 256), f32)
    w1a1 = (w1a1.at[0:256, 0:128].set(W1)
            .at[256:512, 128:256].set(A1) * r2).astype(bf16)
    b1a1 = (jnp.concatenate([b1, a1]) * r2)[None]
    cat2 = jnp.zeros((256, 72), f32)
    cat2 = cat2.at[0:128, 0:64].set(W2 * r2).at[128:256, 64:65].set(A2)
    cat2 = cat2.astype(bf16)
    w3p = jnp.zeros((64, 8), f32).at[:, 0:1].set(W3).astype(bf16)

    grid = B // _TB

    def tiled(shape):
        return pl.BlockSpec(shape, lambda i: (i, 0))

    def whole(x):
        return pl.BlockSpec(x.shape, lambda i: (0,) * x.ndim)

    operands = (
        cards, flat_info.astype(bf16),
        jnp.asarray(_A_CAT, bf16), jnp.asarray(_A_B, bf16),
        jnp.asarray(_SIZES2, bf16), jnp.asarray(_NREQ),
        jnp.asarray(_STRAIGHT, bf16), jnp.asarray(_SEG, bf16),
        jnp.asarray(_NSEG), jnp.asarray(_PM, bf16),
        jnp.asarray(_TLO), jnp.asarray(_THI), jnp.asarray(_CF, bf16),
        w0f2, b0a0, w01, w1a1, b1a1, cat2,
        (b2 * r2)[None], a2[None], w3p, b3[None],
    )
    in_specs = [
        tiled((_TB * _NCARDS, 3)), tiled((_TB, flat_w)),
    ] + [whole(x) for x in operands[2:]]

    out_shape = [jax.ShapeDtypeStruct((B * _MP, 1), f32)] * 2
    out_specs = [pl.BlockSpec((_ROWS, 1), lambda i: (i, 0))] * 2

    logits_col, aux_col = pl.pallas_call(
        _body,
        grid=(grid,),
        in_specs=in_specs,
        out_specs=out_specs,
        out_shape=out_shape,
    )(*operands)

    logits = logits_col.reshape(B, _MP)[:, :_M]
    aux = aux_col.reshape(B, _MP)[:, :_M, None]
    return logits, aux


# speedup vs baseline: 29.4999x; 1.0213x over previous
from itertools import combinations

import numpy as np
import jax
import jax.numpy as jnp
from jax.experimental import pallas as pl

_NCARDS = 8
_M = 218
_MP = 224
_TB = 16
_ROWS = _TB * _MP


def _build_masks():
    rows = []
    for n in range(1, 6):
        for combo in combinations(range(_NCARDS), n):
            m = np.zeros(_NCARDS, dtype=np.float32)
            m[list(combo)] = 1.0
            rows.append(m)
    return np.stack(rows, axis=0)


_MASKS = np.zeros((_MP, _NCARDS), dtype=np.float32)
_MASKS[:_M] = _build_masks()

_A_CAT = np.zeros((_ROWS, 2 * _TB * _NCARDS), dtype=np.float32)
_A_B = np.zeros((_ROWS, _TB), dtype=np.float32)
for _b in range(_TB):
    _r0, _c0 = _b * _MP, _b * _NCARDS
    _A_CAT[_r0:_r0 + _MP, _c0:_c0 + _NCARDS] = _MASKS
    _A_CAT[_r0:_r0 + _MP, _TB * _NCARDS + _c0:_TB * _NCARDS + _c0 + _NCARDS] = 1.0 - _MASKS
    _A_B[_r0:_r0 + _MP, _b] = 1.0

_SIZES2 = np.zeros((_ROWS, 2), dtype=np.float32)
_SIZES2[:, 0] = np.tile(_MASKS.sum(1), _TB)
_SIZES2[:, 1] = _NCARDS - _SIZES2[:, 0]

_STRAIGHT = np.zeros((16, 14), dtype=np.float32)
_STRAIGHT[:10] = np.array([
    [0,1,1,1,1,1,0,0,0,0,0,0,0,0],
    [0,0,1,1,1,1,1,0,0,0,0,0,0,0],
    [0,0,0,1,1,1,1,1,0,0,0,0,0,0],
    [0,0,0,0,1,1,1,1,1,0,0,0,0,0],
    [0,0,0,0,0,1,1,1,1,1,0,0,0,0],
    [0,0,0,0,0,0,1,1,1,1,1,0,0,0],
    [0,0,0,0,0,0,0,1,1,1,1,1,0,0],
    [0,0,0,0,0,0,0,0,1,1,1,1,1,0],
    [0,0,0,0,0,0,0,0,0,1,1,1,1,1],
    [0,1,1,1,1,0,0,0,0,0,0,0,0,1]], dtype=np.float32)

_RC_IN, _SC_IN = range(0, 14), range(14, 20)
_RC_OUT, _SC_OUT = range(20, 34), range(34, 40)
_HI_IN, _HI_OUT = range(40, 56), range(56, 72)
_SC5_IN, _SC5_OUT = 19, 39

_SEG = np.zeros((128, 32), dtype=np.float32)
_NSEG = np.zeros((1, 32), dtype=np.float32)
_PM = np.zeros((384, 32), dtype=np.float32)


def _seg_col(col, lanes):
    for l in lanes:
        _SEG[l, col] = 1.0
    _NSEG[0, col] = len(lanes)


for _half, (_rc, _sc, _hi, _sc5) in enumerate(
        [(_RC_IN, _SC_IN, _HI_IN, _SC5_IN),
         (_RC_OUT, _SC_OUT, _HI_OUT, _SC5_OUT)]):
    _o = 8 * _half
    for _c in (0, 1, 2):
        _seg_col(_o + _c, _rc)
    _seg_col(_o + 3, _sc)
    _PM[_sc5, _o + 3] = 1.0
    _seg_col(_o + 4, _hi)
    for _c in (5, 6):
        for _l in _rc:
            _PM[128 + _l, _o + _c] = 1.0
    for _l in _rc:
        _PM[256 + _l, _o + 7] = 1.0
    _ro = 16 + 4 * _half
    _seg_col(_ro + 0, _rc)
    _seg_col(_ro + 1, _sc)
    _PM[_sc5, _ro + 1] = 1.0
    for _l in _rc:
        _PM[128 + _l, _ro + 2] = 1.0
    _seg_col(_ro + 3, _hi)

_TLO = np.array([[2, 3, 4, 5, 5, 5, 2, 2] * 2], dtype=np.float32)
_THI = np.array([[1e9, 1e9, 1e9, 1e9, 5, 1e9, 2, 2] * 2], dtype=np.float32)

_CF = np.zeros((16, 16), dtype=np.float32)
_FLAG_BASE = [1, 1, 1, 2, 1, 2, 1, 3]
_FLAG_SZREQ = [2, 3, 4, 5, 4, 5, 5, 5]
for _half in range(2):
    _o = 8 * _half
    _CF[_o + 0, _o + 0] = 1.0
    _CF[_o + 1, _o + 1] = 1.0
    _CF[_o + 2, _o + 2] = 1.0
    _CF[_o + 1, _o + 3] = 1.0
    _CF[_o + 2, _o + 3] = -1.0
    _CF[_o + 6, _o + 3] = 1.0
    _CF[_o + 7, _o + 4] = 1.0
    _CF[_o + 4, _o + 5] = 1.0
    _CF[_o + 5, _o + 5] = 1.0
    _CF[_o + 3, _o + 6] = 1.0
    _CF[_o + 4, _o + 7] = 1.0
    _CF[_o + 5, _o + 7] = 1.0
    _CF[_o + 3, _o + 7] = 1.0

_NREQ = np.zeros((_ROWS, 16), dtype=np.float32)
for _j in range(16):
    _sz = _SIZES2[:, _j // 8]
    _NREQ[:, _j] = np.where(_sz >= _FLAG_SZREQ[_j % 8],
                            _FLAG_BASE[_j % 8], 99.0)

_PERM = (list(range(13, 33)) + list(range(46, 66)) +
         list(range(0, 8)) + list(range(33, 41)) +
         [8, 9, 10, 11, 41, 42, 43, 44, 12, 45])

_RSQRT2 = np.float32(1.0 / np.sqrt(2.0))
_HSQRT2 = np.float32(np.sqrt(2.0) / 2.0)


def _gelu_pre(t):
    return (t * 0.7071067811865476) * (1.0 + jax.lax.erf(t))


def _dot(a, b):
    return jnp.dot(a, b, preferred_element_type=jnp.float32)


def _body(cards_ref, flat_ref, acat_ref, ab_ref, sizes_ref, nreq_ref,
          str_ref, seg_ref, nseg_ref, pm_ref, tlo_ref, thi_ref, cf_ref,
          w0f2_ref, b0a0_ref, w01_ref, w1a1_ref, b1a1_ref,
          cat2_ref, b2_ref, a2b_ref, w3_ref, b3_ref, log_ref, aux_ref):
    f32 = jnp.float32
    bf16 = jnp.bfloat16
    cards = cards_ref[...]
    rank = cards[:, 0:1]
    suit = cards[:, 1:2]
    enh = cards[:, 2:3]

    iota14 = jax.lax.broadcasted_iota(jnp.int32, (1, 14), 1)
    iota6 = jax.lax.broadcasted_iota(jnp.int32, (1, 6), 1)
    oh_r = ((rank == iota14) & (iota14 != 0)).astype(bf16)
    oh_s = ((suit == iota6) & (iota6 != 0)).astype(bf16)
    inv = ((rank == 0) & (suit == 0) & (enh == 0)).astype(bf16)
    hits_pre = jax.lax.dot_general(
        oh_r, str_ref[...], (((1,), (1,)), ((), ())),
        preferred_element_type=jnp.float32).astype(bf16)

    nc = _TB * _NCARDS

    def z(n):
        return jnp.zeros((nc, n), bf16)

    row_in = jnp.concatenate(
        [oh_r, oh_s, z(20), hits_pre, z(16), inv, z(55)], axis=-1)
    row_out = jnp.concatenate(
        [z(20), oh_r, oh_s, z(16), hits_pre, z(56)], axis=-1)
    x2 = jnp.concatenate([row_in, row_out], axis=0)

    s_full = _dot(acat_ref[...], x2)

    e = jnp.exp2(4.0 * s_full).astype(bf16)
    p_a = jnp.floor(
        jnp.log2(jnp.maximum(_dot(e, seg_ref[...]) - nseg_ref[...], 1.0))
        * 0.25 + 0.03)
    sb = s_full.astype(bf16)
    q = jnp.concatenate(
        [sb, (s_full > 0).astype(bf16), (s_full >= 2).astype(bf16)],
        axis=-1)
    p = p_a + _dot(q, pm_ref[...])

    v = ((p[:, 0:16] >= tlo_ref[...]) &
         (p[:, 0:16] <= thi_ref[...])).astype(bf16)
    flags = (_dot(v, cf_ref[...]) == nreq_ref[...]).astype(bf16)

    md = jnp.concatenate(
        [sb[:, 0:40], flags, p[:, 16:24].astype(bf16), sizes_ref[...],
         ab_ref[...]], axis=-1)

    flat = flat_ref[...]
    w0dyn = (_dot(flat, w0f2_ref[...]) + b0a0_ref[...]).astype(bf16)
    w01_full = jnp.concatenate([w01_ref[...], w0dyn], axis=0)

    hg0 = _gelu_pre(_dot(md, w01_full).astype(bf16))
    hg1 = _gelu_pre(
        (_dot(hg0, w1a1_ref[...]) + b1a1_ref[...]).astype(bf16))
    t2 = _dot(hg1, cat2_ref[...])
    h2 = _gelu_pre((t2[:, 0:64] + b2_ref[...]).astype(bf16))
    aux = t2[:, 64:65] + a2b_ref[...]
    logit = _dot(h2, w3_ref[...])[:, 0:1] + b3_ref[...]

    valid = s_full[:, 72:73] == 0.0
    log_ref[...] = jnp.where(valid, logit, -1e9)
    aux_ref[...] = jnp.where(valid, aux, 0.0)


def kernel(embeds, flat_info, cards_rank, cards_suit, cards_enhancement,
           W0, b0, W1, b1, W2, b2, W3, b3, A0, a0, A1, a1, A2, a2):
    del embeds
    B = flat_info.shape[0]
    f32 = jnp.float32
    bf16 = jnp.bfloat16
    cards = jnp.stack(
        [cards_rank.astype(jnp.int32), cards_suit.astype(jnp.int32),
         cards_enhancement.astype(jnp.int32)], axis=-1).reshape(B * _NCARDS, 3)

    flat_w = int(flat_info.shape[-1])
    perm = jnp.asarray(np.asarray(_PERM, np.int32))
    r2 = _RSQRT2
    w0md = W0[flat_w:][perm] * r2
    a0md = A0[flat_w:][perm] * r2
    w01 = jnp.concatenate([w0md, a0md], axis=1).astype(bf16)
    w0f2 = (jnp.concatenate([W0[:flat_w], A0[:flat_w]], axis=1)
            * r2).astype(bf16)
    b0a0 = (jnp.concatenate([b0, a0]) * r2)[None]
    w1a1 = jnp.zeros((512, 256), f32)
    w1a1 = (w1a1.at[0:256, 0:128].set(W1)
            .at[256:512, 128:256].set(A1) * r2).astype(bf16)
    b1a1 = (jnp.concatenate([b1, a1]) * r2)[None]
    cat2 = jnp.zeros((256, 72), f32)
    cat2 = cat2.at[0:128, 0:64].set(W2 * r2).at[128:256, 64:65].set(A2)
    cat2 = cat2.astype(bf16)
    w3p = jnp.zeros((64, 8), f32).at[:, 0:1].set(W3).astype(bf16)

    grid = B // _TB

    def tiled(shape):
        return pl.BlockSpec(shape, lambda i: (i, 0))

    def whole(x):
        return pl.BlockSpec(x.shape, lambda i: (0,) * x.ndim)

    operands = (
        cards, flat_info.astype(bf16),
        jnp.asarray(_A_CAT, bf16), jnp.asarray(_A_B, bf16),
        jnp.asarray(_SIZES2, bf16), jnp.asarray(_NREQ),
        jnp.asarray(_STRAIGHT, bf16), jnp.asarray(_SEG, bf16),
        jnp.asarray(_NSEG), jnp.asarray(_PM, bf16),
        jnp.asarray(_TLO), jnp.asarray(_THI), jnp.asarray(_CF, bf16),
        w0f2, b0a0, w01, w1a1, b1a1, cat2,
        (b2 * r2)[None], a2[None], w3p, b3[None],
    )
    in_specs = [
        tiled((_TB * _NCARDS, 3)), tiled((_TB, flat_w)),
    ] + [whole(x) for x in operands[2:]]

    out_shape = [jax.ShapeDtypeStruct((B * _MP, 1), f32)] * 2
    out_specs = [pl.BlockSpec((_ROWS, 1), lambda i: (i, 0))] * 2

    logits_col, aux_col = pl.pallas_call(
        _body,
        grid=(grid,),
        in_specs=in_specs,
        out_specs=out_specs,
        out_shape=out_shape,
    )(*operands)

    logits = logits_col.reshape(B, _MP)[:, :_M]
    aux = aux_col.reshape(B, _MP)[:, :_M, None]
    return logits, aux


# speedup vs baseline: 29.5503x; 1.0017x over previous
from itertools import combinations

import numpy as np
import jax
import jax.numpy as jnp
from jax.experimental import pallas as pl

_NCARDS = 8
_M = 218
_MP = 224
_TB = 16
_ROWS = _TB * _MP


def _build_masks():
    rows = []
    for n in range(1, 6):
        for combo in combinations(range(_NCARDS), n):
            m = np.zeros(_NCARDS, dtype=np.float32)
            m[list(combo)] = 1.0
            rows.append(m)
    return np.stack(rows, axis=0)


_MASKS = np.zeros((_MP, _NCARDS), dtype=np.float32)
_MASKS[:_M] = _build_masks()

_A_CAT = np.zeros((_ROWS, 2 * _TB * _NCARDS), dtype=np.float32)
_A_B = np.zeros((_ROWS, _TB), dtype=np.float32)
for _b in range(_TB):
    _r0, _c0 = _b * _MP, _b * _NCARDS
    _A_CAT[_r0:_r0 + _MP, _c0:_c0 + _NCARDS] = _MASKS
    _A_CAT[_r0:_r0 + _MP, _TB * _NCARDS + _c0:_TB * _NCARDS + _c0 + _NCARDS] = 1.0 - _MASKS
    _A_B[_r0:_r0 + _MP, _b] = 1.0

_SIZES2 = np.zeros((_ROWS, 2), dtype=np.float32)
_SIZES2[:, 0] = np.tile(_MASKS.sum(1), _TB)
_SIZES2[:, 1] = _NCARDS - _SIZES2[:, 0]

_STRAIGHT = np.zeros((16, 14), dtype=np.float32)
_STRAIGHT[:10] = np.array([
    [0,1,1,1,1,1,0,0,0,0,0,0,0,0],
    [0,0,1,1,1,1,1,0,0,0,0,0,0,0],
    [0,0,0,1,1,1,1,1,0,0,0,0,0,0],
    [0,0,0,0,1,1,1,1,1,0,0,0,0,0],
    [0,0,0,0,0,1,1,1,1,1,0,0,0,0],
    [0,0,0,0,0,0,1,1,1,1,1,0,0,0],
    [0,0,0,0,0,0,0,1,1,1,1,1,0,0],
    [0,0,0,0,0,0,0,0,1,1,1,1,1,0],
    [0,0,0,0,0,0,0,0,0,1,1,1,1,1],
    [0,1,1,1,1,0,0,0,0,0,0,0,0,1]], dtype=np.float32)

_RC_IN, _SC_IN = range(0, 14), range(14, 20)
_RC_OUT, _SC_OUT = range(20, 34), range(34, 40)
_HI_IN, _HI_OUT = range(40, 56), range(56, 72)
_SC5_IN, _SC5_OUT = 19, 39

_SEG = np.zeros((128, 32), dtype=np.float32)
_NSEG = np.zeros((1, 32), dtype=np.float32)
_PM = np.zeros((384, 32), dtype=np.float32)
_TLO = np.full((1, 32), 1e9, dtype=np.float32)


def _seg_col(col, lanes):
    for l in lanes:
        _SEG[l, col] = 1.0
    _NSEG[0, col] = len(lanes)


for _half, (_rc, _sc, _hi, _sc5) in enumerate(
        [(_RC_IN, _SC_IN, _HI_IN, _SC5_IN),
         (_RC_OUT, _SC_OUT, _HI_OUT, _SC5_OUT)]):
    _ro = 4 * _half
    _seg_col(_ro + 0, _rc)
    _seg_col(_ro + 1, _sc)
    _PM[_sc5, _ro + 1] = 1.0
    for _l in _rc:
        _PM[128 + _l, _ro + 2] = 1.0
    _seg_col(_ro + 3, _hi)
    _o = 8 + 11 * _half
    for _c, _th in ((0, 2), (1, 3), (2, 4)):
        _seg_col(_o + _c, _rc)
        _TLO[0, _o + _c] = _th
    _seg_col(_o + 3, _sc)
    _PM[_sc5, _o + 3] = 1.0
    _TLO[0, _o + 3] = 5
    for _c, _th in ((4, 5), (5, 6)):
        _seg_col(_o + _c, _hi)
        _TLO[0, _o + _c] = _th
    for _c, _th in ((6, 2), (7, 3), (8, 5)):
        for _l in _rc:
            _PM[128 + _l, _o + _c] = 1.0
        _TLO[0, _o + _c] = _th
    for _c, _th in ((9, 2), (10, 3)):
        for _l in _rc:
            _PM[256 + _l, _o + _c] = 1.0
        _TLO[0, _o + _c] = _th

_CF = np.zeros((32, 16), dtype=np.float32)
_FLAG_BASE = [1, 1, 1, 2, 1, 2, 1, 3]
_FLAG_SZREQ = [2, 3, 4, 5, 4, 5, 5, 5]
for _half in range(2):
    _o = 8 + 11 * _half
    _fo = 8 * _half
    _CF[_o + 0, _fo + 0] = 1.0
    _CF[_o + 1, _fo + 1] = 1.0
    _CF[_o + 2, _fo + 2] = 1.0
    _CF[_o + 1, _fo + 3] = 1.0
    _CF[_o + 2, _fo + 3] = -1.0
    _CF[_o + 6, _fo + 3] = 1.0
    _CF[_o + 7, _fo + 3] = -1.0
    _CF[_o + 9, _fo + 4] = 1.0
    _CF[_o + 10, _fo + 4] = -1.0
    _CF[_o + 4, _fo + 5] = 1.0
    _CF[_o + 5, _fo + 5] = -1.0
    _CF[_o + 8, _fo + 5] = 1.0
    _CF[_o + 3, _fo + 6] = 1.0
    _CF[_o + 4, _fo + 7] = 1.0
    _CF[_o + 5, _fo + 7] = -1.0
    _CF[_o + 8, _fo + 7] = 1.0
    _CF[_o + 3, _fo + 7] = 1.0

_NREQ = np.zeros((_ROWS, 16), dtype=np.float32)
for _j in range(16):
    _sz = _SIZES2[:, _j // 8]
    _NREQ[:, _j] = np.where(_sz >= _FLAG_SZREQ[_j % 8],
                            _FLAG_BASE[_j % 8], 99.0)

_PERM = (list(range(13, 33)) + list(range(46, 66)) +
         list(range(0, 8)) + list(range(33, 41)) +
         [8, 9, 10, 11, 41, 42, 43, 44, 12, 45])

_RSQRT2 = np.float32(1.0 / np.sqrt(2.0))
_HSQRT2 = np.float32(np.sqrt(2.0) / 2.0)


def _gelu_pre(t):
    return (t * 0.7071067811865476) * (1.0 + jax.lax.erf(t))


def _dot(a, b):
    return jnp.dot(a, b, preferred_element_type=jnp.float32)


def _body(cards_ref, flat_ref, acat_ref, ab_ref, sizes_ref, nreq_ref,
          str_ref, seg_ref, nseg_ref, pm_ref, tlo_ref, cf_ref,
          w0f2_ref, b0a0_ref, w01_ref, w1a1_ref, b1a1_ref,
          cat2_ref, b2_ref, a2b_ref, w3_ref, b3_ref, log_ref, aux_ref):
    f32 = jnp.float32
    bf16 = jnp.bfloat16
    cards = cards_ref[...]
    rank = cards[:, 0:1]
    suit = cards[:, 1:2]
    enh = cards[:, 2:3]

    iota14 = jax.lax.broadcasted_iota(jnp.int32, (1, 14), 1)
    iota6 = jax.lax.broadcasted_iota(jnp.int32, (1, 6), 1)
    oh_r = ((rank == iota14) & (iota14 != 0)).astype(bf16)
    oh_s = ((suit == iota6) & (iota6 != 0)).astype(bf16)
    inv = ((rank == 0) & (suit == 0) & (enh == 0)).astype(bf16)
    hits_pre = jax.lax.dot_general(
        oh_r, str_ref[...], (((1,), (1,)), ((), ())),
        preferred_element_type=jnp.float32).astype(bf16)

    nc = _TB * _NCARDS

    def z(n):
        return jnp.zeros((nc, n), bf16)

    row_in = jnp.concatenate(
        [oh_r, oh_s, z(20), hits_pre, z(16), inv, z(55)], axis=-1)
    row_out = jnp.concatenate(
        [z(20), oh_r, oh_s, z(16), hits_pre, z(56)], axis=-1)
    x2 = jnp.concatenate([row_in, row_out], axis=0)

    s_full = _dot(acat_ref[...], x2)

    e = jnp.exp2(4.0 * s_full).astype(bf16)
    p_a = jnp.floor(
        jnp.log2(jnp.maximum(_dot(e, seg_ref[...]) - nseg_ref[...], 1.0))
        * 0.25 + 0.03)
    sb = s_full.astype(bf16)
    q = jnp.concatenate(
        [sb, (s_full > 0).astype(bf16), (s_full >= 2).astype(bf16)],
        axis=-1)
    p = p_a + _dot(q, pm_ref[...])

    v = (p >= tlo_ref[...]).astype(bf16)
    flags = (_dot(v, cf_ref[...]) == nreq_ref[...]).astype(bf16)

    md = jnp.concatenate(
        [sb[:, 0:40], flags, p[:, 0:8].astype(bf16), sizes_ref[...],
         ab_ref[...]], axis=-1)

    flat = flat_ref[...]
    w0dyn = (_dot(flat, w0f2_ref[...]) + b0a0_ref[...]).astype(bf16)
    w01_full = jnp.concatenate([w01_ref[...], w0dyn], axis=0)

    hg0 = _gelu_pre(_dot(md, w01_full).astype(bf16))
    hg1 = _gelu_pre(
        (_dot(hg0, w1a1_ref[...]) + b1a1_ref[...]).astype(bf16))
    t2 = _dot(hg1, cat2_ref[...])
    h2 = _gelu_pre((t2[:, 0:64] + b2_ref[...]).astype(bf16))
    aux = t2[:, 64:65] + a2b_ref[...]
    logit = _dot(h2, w3_ref[...])[:, 0:1] + b3_ref[...]

    valid = s_full[:, 72:73] == 0.0
    log_ref[...] = jnp.where(valid, logit, -1e9)
    aux_ref[...] = jnp.where(valid, aux, 0.0)


def kernel(embeds, flat_info, cards_rank, cards_suit, cards_enhancement,
           W0, b0, W1, b1, W2, b2, W3, b3, A0, a0, A1, a1, A2, a2):
    del embeds
    B = flat_info.shape[0]
    f32 = jnp.float32
    bf16 = jnp.bfloat16
    cards = jnp.stack(
        [cards_rank.astype(jnp.int32), cards_suit.astype(jnp.int32),
         cards_enhancement.astype(jnp.int32)], axis=-1).reshape(B * _NCARDS, 3)

    flat_w = int(flat_info.shape[-1])
    perm = jnp.asarray(np.asarray(_PERM, np.int32))
    r2 = _RSQRT2
    w0md = W0[flat_w:][perm] * r2
    a0md = A0[flat_w:][perm] * r2
    w01 = jnp.concatenate([w0md, a0md], axis=1).astype(bf16)
    w0f2 = (jnp.concatenate([W0[:flat_w], A0[:flat_w]], axis=1)
            * r2).astype(bf16)
    b0a0 = (jnp.concatenate([b0, a0]) * r2)[None]
    w1a1 = jnp.zeros((512, 256), f32)
    w1a1 = (w1a1.at[0:256, 0:128].set(W1)
            .at[256:512, 128:256].set(A1) * r2).astype(bf16)
    b1a1 = (jnp.concatenate([b1, a1]) * r2)[None]
    cat2 = jnp.zeros((256, 72), f32)
    cat2 = cat2.at[0:128, 0:64].set(W2 * r2).at[128:256, 64:65].set(A2)
    cat2 = cat2.astype(bf16)
    w3p = jnp.zeros((64, 8), f32).at[:, 0:1].set(W3).astype(bf16)

    grid = B // _TB

    def tiled(shape):
        return pl.BlockSpec(shape, lambda i: (i, 0))

    def whole(x):
        return pl.BlockSpec(x.shape, lambda i: (0,) * x.ndim)

    operands = (
        cards, flat_info.astype(bf16),
        jnp.asarray(_A_CAT, bf16), jnp.asarray(_A_B, bf16),
        jnp.asarray(_SIZES2, bf16), jnp.asarray(_NREQ),
        jnp.asarray(_STRAIGHT, bf16), jnp.asarray(_SEG, bf16),
        jnp.asarray(_NSEG), jnp.asarray(_PM, bf16),
        jnp.asarray(_TLO), jnp.asarray(_CF, bf16),
        w0f2, b0a0, w01, w1a1, b1a1, cat2,
        (b2 * r2)[None], a2[None], w3p, b3[None],
    )
    in_specs = [
        tiled((_TB * _NCARDS, 3)), tiled((_TB, flat_w)),
    ] + [whole(x) for x in operands[2:]]

    out_shape = [jax.ShapeDtypeStruct((B * _MP, 1), f32)] * 2
    out_specs = [pl.BlockSpec((_ROWS, 1), lambda i: (i, 0))] * 2

    logits_col, aux_col = pl.pallas_call(
        _body,
        grid=(grid,),
        in_specs=in_specs,
        out_specs=out_specs,
        out_shape=out_shape,
    )(*operands)

    logits = logits_col.reshape(B, _MP)[:, :_M]
    aux = aux_col.reshape(B, _MP)[:, :_M, None]
    return logits, aux
